# Initial kernel scaffold; baseline (speedup 1.0000x reference)
import functools

import jax
import jax.numpy as jnp
from jax import lax
from jax.experimental import pallas as pl
from jax.experimental.pallas import tpu as pltpu
from jax.experimental.pallas import tpu_sc as plsc

N = 50000
E = 800000
D_IN = 128
H = 64
HH = H // 2
G = 256
C = 29

CHUNK = 128
NROWS = 6272
EP = NROWS * CHUNK
ROWS_DEG = NROWS // 32
ROWS_AGG = NROWS // 16
N_ACC = 51200
STRIPE = N_ACC // 16
LAST_ROWS = N - 15 * STRIPE

BLK = 2000
NBLK = N // BLK
EPS = 1e-5


def _sc_mesh():
    return plsc.VectorSubcoreMesh(core_axis_name="c", subcore_axis_name="s")


def _sc_degree(dst2d, ones32, zstripe):
    out_t = [jax.ShapeDtypeStruct((N, HH), jnp.float32)] * 2

    @functools.partial(
        pl.kernel, mesh=_sc_mesh(), out_type=out_t,
        scratch_types=[
            pltpu.VMEM_SHARED((N_ACC, HH), jnp.float32),
            pltpu.VMEM((CHUNK, HH), jnp.float32),
            pltpu.VMEM((CHUNK,), jnp.int32),
        ],
    )
    def k(dst_hbm, ones_hbm, z_hbm, cnt0_hbm, cnt1_hbm, acc, onesv, idxv):
        c = lax.axis_index("c")
        s = lax.axis_index("s")
        pltpu.sync_copy(ones_hbm, onesv)
        pltpu.sync_copy(z_hbm, acc.at[pl.ds(s * STRIPE, STRIPE)])
        plsc.subcore_barrier()
        base = (c * 16 + s) * ROWS_DEG

        @pl.loop(0, ROWS_DEG)
        def _(j):
            pltpu.sync_copy(dst_hbm.at[base + j], idxv)
            pltpu.sync_copy(onesv, acc.at[idxv], add=True)

        plsc.subcore_barrier()
        r0 = s * STRIPE

        @pl.when(c == 0)
        def _():
            @pl.when(s < 15)
            def _():
                pltpu.sync_copy(acc.at[pl.ds(r0, STRIPE)],
                                cnt0_hbm.at[pl.ds(r0, STRIPE)])
            @pl.when(s == 15)
            def _():
                pltpu.sync_copy(acc.at[pl.ds(15 * STRIPE, LAST_ROWS)],
                                cnt0_hbm.at[pl.ds(15 * STRIPE, LAST_ROWS)])

        @pl.when(c == 1)
        def _():
            @pl.when(s < 15)
            def _():
                pltpu.sync_copy(acc.at[pl.ds(r0, STRIPE)],
                                cnt1_hbm.at[pl.ds(r0, STRIPE)])
            @pl.when(s == 15)
            def _():
                pltpu.sync_copy(acc.at[pl.ds(15 * STRIPE, LAST_ROWS)],
                                cnt1_hbm.at[pl.ds(15 * STRIPE, LAST_ROWS)])

    return k(dst2d, ones32, zstripe)


def _sc_aggregate(hs_lo, hs_hi, src2d, dst2d, zstripe):
    out_t = [jax.ShapeDtypeStruct((N, HH), jnp.float32)] * 2

    @functools.partial(
        pl.kernel, mesh=_sc_mesh(), out_type=out_t,
        scratch_types=[
            pltpu.VMEM_SHARED((N_ACC, HH), jnp.float32),
            pltpu.VMEM((CHUNK, HH), jnp.float32),
            pltpu.VMEM((CHUNK,), jnp.int32),
            pltpu.VMEM((CHUNK,), jnp.int32),
        ],
    )
    def k(hslo_hbm, hshi_hbm, src_hbm, dst_hbm, z_hbm,
          outlo_hbm, outhi_hbm, acc, msg, sidx, didx):
        c = lax.axis_index("c")
        s = lax.axis_index("s")
        pltpu.sync_copy(z_hbm, acc.at[pl.ds(s * STRIPE, STRIPE)])
        plsc.subcore_barrier()
        base = s * ROWS_AGG

        @pl.loop(0, ROWS_AGG)
        def _(j):
            pltpu.sync_copy(src_hbm.at[base + j], sidx)
            pltpu.sync_copy(dst_hbm.at[base + j], didx)

            @pl.when(c == 0)
            def _():
                pltpu.sync_copy(hslo_hbm.at[sidx], msg)

            @pl.when(c == 1)
            def _():
                pltpu.sync_copy(hshi_hbm.at[sidx], msg)

            pltpu.sync_copy(msg, acc.at[didx], add=True)

        plsc.subcore_barrier()
        r0 = s * STRIPE

        @pl.when(c == 0)
        def _():
            @pl.when(s < 15)
            def _():
                pltpu.sync_copy(acc.at[pl.ds(r0, STRIPE)],
                                outlo_hbm.at[pl.ds(r0, STRIPE)])
            @pl.when(s == 15)
            def _():
                pltpu.sync_copy(acc.at[pl.ds(15 * STRIPE, LAST_ROWS)],
                                outlo_hbm.at[pl.ds(15 * STRIPE, LAST_ROWS)])

        @pl.when(c == 1)
        def _():
            @pl.when(s < 15)
            def _():
                pltpu.sync_copy(acc.at[pl.ds(r0, STRIPE)],
                                outhi_hbm.at[pl.ds(r0, STRIPE)])
            @pl.when(s == 15)
            def _():
                pltpu.sync_copy(acc.at[pl.ds(15 * STRIPE, LAST_ROWS)],
                                outhi_hbm.at[pl.ds(15 * STRIPE, LAST_ROWS)])

    return k(hs_lo, hs_hi, src2d, dst2d, zstripe)


def _front(x, cnt0, cnt1, winT, b_in, w1aT, w1bT):
    def body(x_ref, c0_ref, c1_ref, w_ref, b_ref, wa_ref, wb_ref,
             lo_ref, hi_ref, d_ref):
        deg = 1.0 + (jnp.sum(c0_ref[...], axis=1, keepdims=True)
                     + jnp.sum(c1_ref[...], axis=1, keepdims=True)) * (1.0 / HH)
        d = lax.rsqrt(deg)
        h0 = jnp.maximum(
            jnp.dot(x_ref[...], w_ref[...],
                    preferred_element_type=jnp.float32) + b_ref[...], 0.0)
        lo_ref[...] = jnp.dot(h0, wa_ref[...],
                              preferred_element_type=jnp.float32) * d
        hi_ref[...] = jnp.dot(h0, wb_ref[...],
                              preferred_element_type=jnp.float32) * d
        d_ref[...] = d

    full = lambda shape: pl.BlockSpec(shape, lambda i: (0, 0))
    return pl.pallas_call(
        body,
        grid=(NBLK,),
        in_specs=[
            pl.BlockSpec((BLK, D_IN), lambda i: (i, 0)),
            pl.BlockSpec((BLK, HH), lambda i: (i, 0)),
            pl.BlockSpec((BLK, HH), lambda i: (i, 0)),
            full((D_IN, H)), full((1, H)), full((H, HH)), full((H, HH)),
        ],
        out_specs=[
            pl.BlockSpec((BLK, HH), lambda i: (i, 0)),
            pl.BlockSpec((BLK, HH), lambda i: (i, 0)),
            pl.BlockSpec((BLK, 1), lambda i: (i, 0)),
        ],
        out_shape=[
            jax.ShapeDtypeStruct((N, HH), jnp.float32),
            jax.ShapeDtypeStruct((N, HH), jnp.float32),
            jax.ShapeDtypeStruct((N, 1), jnp.float32),
        ],
    )(x, cnt0, cnt1, winT, b_in, w1aT, w1bT)


def _bn_stats(s_lo, s_hi, hs_lo, hs_hi, dis, b_lo, b_hi):
    def body(sl_ref, sh_ref, hl_ref, hh_ref, d_ref, bl_ref, bh_ref,
             s1l_ref, s1h_ref, s2l_ref, s2h_ref):
        @pl.when(pl.program_id(0) == 0)
        def _():
            s1l_ref[...] = jnp.zeros_like(s1l_ref)
            s1h_ref[...] = jnp.zeros_like(s1h_ref)
            s2l_ref[...] = jnp.zeros_like(s2l_ref)
            s2h_ref[...] = jnp.zeros_like(s2h_ref)

        d = d_ref[...]
        tl = (sl_ref[...] + hl_ref[...]) * d + bl_ref[...]
        th = (sh_ref[...] + hh_ref[...]) * d + bh_ref[...]
        s1l_ref[...] += jnp.sum(tl, axis=0, keepdims=True)
        s1h_ref[...] += jnp.sum(th, axis=0, keepdims=True)
        s2l_ref[...] += jnp.sum(tl * tl, axis=0, keepdims=True)
        s2h_ref[...] += jnp.sum(th * th, axis=0, keepdims=True)

    blk = lambda: pl.BlockSpec((BLK, HH), lambda i: (i, 0))
    acc = lambda: pl.BlockSpec((1, HH), lambda i: (0, 0))
    return pl.pallas_call(
        body,
        grid=(NBLK,),
        in_specs=[blk(), blk(), blk(), blk(),
                  pl.BlockSpec((BLK, 1), lambda i: (i, 0)),
                  acc(), acc()],
        out_specs=[acc(), acc(), acc(), acc()],
        out_shape=[jax.ShapeDtypeStruct((1, HH), jnp.float32)] * 4,
    )(s_lo, s_hi, hs_lo, hs_hi, dis, b_lo, b_hi)


def _bn_apply(s_lo, s_hi, hs_lo, hs_hi, dis, b_lo, b_hi, g_lo, g_hi,
              be_lo, be_hi, stats, wq, last):
    s1l, s1h, s2l, s2h = stats

    def body(sl_ref, sh_ref, hl_ref, hh_ref, d_ref, bl_ref, bh_ref,
             gl_ref, gh_ref, bel_ref, beh_ref,
             s1l_ref, s1h_ref, s2l_ref, s2h_ref, *rest):
        if last:
            ol_ref, oh_ref = rest
        else:
            wll_ref, wlh_ref, whl_ref, whh_ref, ol_ref, oh_ref = rest
        d = d_ref[...]
        tl = (sl_ref[...] + hl_ref[...]) * d + bl_ref[...]
        th = (sh_ref[...] + hh_ref[...]) * d + bh_ref[...]
        inv_n = 1.0 / N
        ml = s1l_ref[...] * inv_n
        mh = s1h_ref[...] * inv_n
        vl = s2l_ref[...] * inv_n - ml * ml
        vh = s2h_ref[...] * inv_n - mh * mh
        scl = gl_ref[...] * lax.rsqrt(vl + EPS)
        sch = gh_ref[...] * lax.rsqrt(vh + EPS)
        rl = jnp.maximum(tl * scl + (bel_ref[...] - ml * scl), 0.0)
        rh = jnp.maximum(th * sch + (beh_ref[...] - mh * sch), 0.0)
        if last:
            ol_ref[...] = rl
            oh_ref[...] = rh
        else:
            ol_ref[...] = (jnp.dot(rl, wll_ref[...],
                                   preferred_element_type=jnp.float32)
                           + jnp.dot(rh, wlh_ref[...],
                                     preferred_element_type=jnp.float32)) * d
            oh_ref[...] = (jnp.dot(rl, whl_ref[...],
                                   preferred_element_type=jnp.float32)
                           + jnp.dot(rh, whh_ref[...],
                                     preferred_element_type=jnp.float32)) * d

    blk = lambda: pl.BlockSpec((BLK, HH), lambda i: (i, 0))
    row = lambda: pl.BlockSpec((1, HH), lambda i: (0, 0))
    wspec = lambda: pl.BlockSpec((HH, HH), lambda i: (0, 0))
    in_specs = [blk(), blk(), blk(), blk(),
                pl.BlockSpec((BLK, 1), lambda i: (i, 0)),
                row(), row(), row(), row(), row(), row(),
                row(), row(), row(), row()]
    args = [s_lo, s_hi, hs_lo, hs_hi, dis, b_lo, b_hi, g_lo, g_hi,
            be_lo, be_hi, s1l, s1h, s2l, s2h]
    if not last:
        in_specs += [wspec()] * 4
        args += list(wq)
    return pl.pallas_call(
        body,
        grid=(NBLK,),
        in_specs=in_specs,
        out_specs=[blk(), blk()],
        out_shape=[jax.ShapeDtypeStruct((N, HH), jnp.float32)] * 2,
    )(*args)


def _pool(h_lo, h_hi, starts):
    CH = 64

    def body(st_ref, hl_ref, hh_ref, ml_ref, mh_ref, xl_ref, xh_ref):
        g = pl.program_id(0)
        s0 = st_ref[g]
        e0 = st_ref[g + 1]
        n = e0 - s0
        nch = lax.div(n + (CH - 1), CH)

        def step(i, carry):
            sml, smh, mxl, mxh = carry
            base = s0 + i * CH
            b = jnp.minimum(base, N - CH)
            rl = hl_ref[pl.ds(b, CH), :]
            rh = hh_ref[pl.ds(b, CH), :]
            ridx = b + lax.broadcasted_iota(jnp.int32, (CH, 1), 0)
            m = (ridx >= base) & (ridx < e0)
            sml = sml + jnp.sum(jnp.where(m, rl, 0.0), axis=0, keepdims=True)
            smh = smh + jnp.sum(jnp.where(m, rh, 0.0), axis=0, keepdims=True)
            mxl = jnp.maximum(mxl, jnp.max(jnp.where(m, rl, -jnp.inf),
                                           axis=0, keepdims=True))
            mxh = jnp.maximum(mxh, jnp.max(jnp.where(m, rh, -jnp.inf),
                                           axis=0, keepdims=True))
            return sml, smh, mxl, mxh

        z = jnp.zeros((1, HH), jnp.float32)
        ninf = jnp.full((1, HH), -jnp.inf, jnp.float32)
        sml, smh, mxl, mxh = lax.fori_loop(0, nch, step, (z, z, ninf, ninf))
        cnt = jnp.maximum(n.astype(jnp.float32), 1.0)
        ml_ref[...] = sml / cnt
        mh_ref[...] = smh / cnt
        xl_ref[...] = jnp.where(n > 0, mxl, 0.0)
        xh_ref[...] = jnp.where(n > 0, mxh, 0.0)

    grid_spec = pltpu.PrefetchScalarGridSpec(
        num_scalar_prefetch=1,
        grid=(G,),
        in_specs=[pl.BlockSpec((N, HH), lambda g, st: (0, 0)),
                  pl.BlockSpec((N, HH), lambda g, st: (0, 0))],
        out_specs=[pl.BlockSpec((1, HH), lambda g, st: (g, 0))] * 4,
    )
    return pl.pallas_call(
        body,
        grid_spec=grid_spec,
        out_shape=[jax.ShapeDtypeStruct((G, HH), jnp.float32)] * 4,
    )(starts, h_lo, h_hi)


def _mlp(pooled, wc1T, bc1, wc2T, bc2):
    p0, p1, p2, p3 = pooled
    w0, w1, w2, w3 = wc1T

    def body(p0_ref, p1_ref, p2_ref, p3_ref, w0_ref, w1_ref, w2_ref, w3_ref,
             b1_ref, wc2_ref, b2_ref, o_ref):
        hc = (jnp.dot(p0_ref[...], w0_ref[...], preferred_element_type=jnp.float32)
              + jnp.dot(p1_ref[...], w1_ref[...], preferred_element_type=jnp.float32)
              + jnp.dot(p2_ref[...], w2_ref[...], preferred_element_type=jnp.float32)
              + jnp.dot(p3_ref[...], w3_ref[...], preferred_element_type=jnp.float32))
        hc = jnp.maximum(hc + b1_ref[...], 0.0)
        o_ref[...] = jnp.dot(hc, wc2_ref[...],
                             preferred_element_type=jnp.float32) + b2_ref[...]

    return pl.pallas_call(
        body,
        out_shape=jax.ShapeDtypeStruct((G, C), jnp.float32),
    )(p0, p1, p2, p3, w0, w1, w2, w3, bc1, wc2T, bc2)


def kernel(x, edge_index, batch, W_in, b_in, W1, b1, g1, be1, W2, b2, g2,
           be2, W3, b3, g3, be3, Wc1, bc1, Wc2, bc2):
    f32 = jnp.float32
    src = edge_index[0]
    dst = edge_index[1]
    pad = EP - E
    src2d = jnp.concatenate(
        [src, jnp.zeros((pad,), src.dtype)]).reshape(NROWS, CHUNK)
    dst2d = jnp.concatenate(
        [dst, jnp.full((pad,), N, dst.dtype)]).reshape(NROWS, CHUNK)
    ones32 = jnp.ones((CHUNK, HH), f32)
    zstripe = jnp.zeros((STRIPE, HH), f32)
    starts = jnp.searchsorted(
        batch, jnp.arange(G + 1, dtype=batch.dtype)).astype(jnp.int32)

    halves = lambda v: (v[:HH].reshape(1, HH).astype(f32),
                        v[HH:].reshape(1, HH).astype(f32))
    quarters = lambda W: (W[:HH, :HH].T, W[:HH, HH:].T,
                          W[HH:, :HH].T, W[HH:, HH:].T)
    winT = W_in.T
    b_in2 = b_in.reshape(1, H)
    w1aT, w1bT = W1[:HH, :].T, W1[HH:, :].T
    b1l, b1h = halves(b1); g1l, g1h = halves(g1); be1l, be1h = halves(be1)
    b2l, b2h = halves(b2); g2l, g2h = halves(g2); be2l, be2h = halves(be2)
    b3l, b3h = halves(b3); g3l, g3h = halves(g3); be3l, be3h = halves(be3)
    wq2 = quarters(W2)
    wq3 = quarters(W3)
    wc1T = tuple(Wc1[:, i * HH:(i + 1) * HH].T for i in range(4))
    bc1r = bc1.reshape(1, H)
    wc2T = Wc2.T
    bc2r = bc2.reshape(1, C)

    cnt0, cnt1 = _sc_degree(dst2d, ones32, zstripe)
    hs_lo, hs_hi, dis = _front(x, cnt0, cnt1, winT, b_in2, w1aT, w1bT)

    s_lo, s_hi = _sc_aggregate(hs_lo, hs_hi, src2d, dst2d, zstripe)
    st = _bn_stats(s_lo, s_hi, hs_lo, hs_hi, dis, b1l, b1h)
    hs_lo, hs_hi = _bn_apply(s_lo, s_hi, hs_lo, hs_hi, dis, b1l, b1h,
                             g1l, g1h, be1l, be1h, st, wq2, last=False)
    s_lo, s_hi = _sc_aggregate(hs_lo, hs_hi, src2d, dst2d, zstripe)
    st = _bn_stats(s_lo, s_hi, hs_lo, hs_hi, dis, b2l, b2h)
    hs_lo, hs_hi = _bn_apply(s_lo, s_hi, hs_lo, hs_hi, dis, b2l, b2h,
                             g2l, g2h, be2l, be2h, st, wq3, last=False)
    s_lo, s_hi = _sc_aggregate(hs_lo, hs_hi, src2d, dst2d, zstripe)
    st = _bn_stats(s_lo, s_hi, hs_lo, hs_hi, dis, b3l, b3h)
    h3_lo, h3_hi = _bn_apply(s_lo, s_hi, hs_lo, hs_hi, dis, b3l, b3h,
                             g3l, g3h, be3l, be3h, st, None, last=True)

    m_lo, m_hi, x_lo, x_hi = _pool(h3_lo, h3_hi, starts)
    return _mlp((m_lo, m_hi, x_lo, x_hi), wc1T, bc1r, wc2T, bc2r)


# speedup vs baseline: 9.7551x; 9.7551x over previous
import functools

import jax
import jax.numpy as jnp
from jax import lax
from jax.experimental import pallas as pl
from jax.experimental.pallas import tpu as pltpu
from jax.experimental.pallas import tpu_sc as plsc

N = 50000
E = 800000
D_IN = 128
H = 64
HH = H // 2
G = 256
C = 29

CHUNK = 128
NROWS = 6272
EP = NROWS * CHUNK
ROWS_DEG = NROWS // 32
ROWS_AGG = NROWS // 16
N_ACC = 51200
STRIPE = N_ACC // 16
LAST_ROWS = N - 15 * STRIPE

BLK = 2000
NBLK = N // BLK
EPS = 1e-5


def _sc_mesh():
    return plsc.VectorSubcoreMesh(core_axis_name="c", subcore_axis_name="s")


_SC_PARAMS = pltpu.CompilerParams(use_tc_tiling_on_sc=False)


def _sc_degree(dst2d, ones32, zstripe):
    out_t = [jax.ShapeDtypeStruct((N, HH), jnp.float32)] * 2

    @functools.partial(
        pl.kernel, mesh=_sc_mesh(), out_type=out_t,
        compiler_params=_SC_PARAMS,
        scratch_types=[
            pltpu.VMEM_SHARED((N_ACC, HH), jnp.float32),
            pltpu.VMEM((CHUNK, HH), jnp.float32),
            pltpu.VMEM((CHUNK,), jnp.int32),
        ],
    )
    def k(dst_hbm, ones_hbm, z_hbm, cnt0_hbm, cnt1_hbm, acc, onesv, idxv):
        c = lax.axis_index("c")
        s = lax.axis_index("s")
        pltpu.sync_copy(ones_hbm, onesv)
        pltpu.sync_copy(z_hbm, acc.at[pl.ds(s * STRIPE, STRIPE)])
        plsc.subcore_barrier()
        base = (c * 16 + s) * ROWS_DEG

        @pl.loop(0, ROWS_DEG)
        def _(j):
            pltpu.sync_copy(dst_hbm.at[base + j], idxv)
            pltpu.sync_copy(onesv, acc.at[idxv], add=True)

        plsc.subcore_barrier()
        r0 = s * STRIPE

        @pl.when(c == 0)
        def _():
            @pl.when(s < 15)
            def _():
                pltpu.sync_copy(acc.at[pl.ds(r0, STRIPE)],
                                cnt0_hbm.at[pl.ds(r0, STRIPE)])
            @pl.when(s == 15)
            def _():
                pltpu.sync_copy(acc.at[pl.ds(15 * STRIPE, LAST_ROWS)],
                                cnt0_hbm.at[pl.ds(15 * STRIPE, LAST_ROWS)])

        @pl.when(c == 1)
        def _():
            @pl.when(s < 15)
            def _():
                pltpu.sync_copy(acc.at[pl.ds(r0, STRIPE)],
                                cnt1_hbm.at[pl.ds(r0, STRIPE)])
            @pl.when(s == 15)
            def _():
                pltpu.sync_copy(acc.at[pl.ds(15 * STRIPE, LAST_ROWS)],
                                cnt1_hbm.at[pl.ds(15 * STRIPE, LAST_ROWS)])

    return k(dst2d, ones32, zstripe)


def _sc_aggregate(hs_lo, hs_hi, src2d, dst2d, zstripe):
    out_t = [jax.ShapeDtypeStruct((N, HH), jnp.float32)] * 2

    @functools.partial(
        pl.kernel, mesh=_sc_mesh(), out_type=out_t,
        compiler_params=_SC_PARAMS,
        scratch_types=[
            pltpu.VMEM_SHARED((N_ACC, HH), jnp.float32),
            pltpu.VMEM((CHUNK, HH), jnp.float32),
            pltpu.VMEM((CHUNK,), jnp.int32),
            pltpu.VMEM((CHUNK,), jnp.int32),
        ],
    )
    def k(hslo_hbm, hshi_hbm, src_hbm, dst_hbm, z_hbm,
          outlo_hbm, outhi_hbm, acc, msg, sidx, didx):
        c = lax.axis_index("c")
        s = lax.axis_index("s")
        pltpu.sync_copy(z_hbm, acc.at[pl.ds(s * STRIPE, STRIPE)])
        plsc.subcore_barrier()
        base = s * ROWS_AGG

        @pl.loop(0, ROWS_AGG)
        def _(j):
            pltpu.sync_copy(src_hbm.at[base + j], sidx)
            pltpu.sync_copy(dst_hbm.at[base + j], didx)

            @pl.when(c == 0)
            def _():
                pltpu.sync_copy(hslo_hbm.at[sidx], msg)

            @pl.when(c == 1)
            def _():
                pltpu.sync_copy(hshi_hbm.at[sidx], msg)

            pltpu.sync_copy(msg, acc.at[didx], add=True)

        plsc.subcore_barrier()
        r0 = s * STRIPE

        @pl.when(c == 0)
        def _():
            @pl.when(s < 15)
            def _():
                pltpu.sync_copy(acc.at[pl.ds(r0, STRIPE)],
                                outlo_hbm.at[pl.ds(r0, STRIPE)])
            @pl.when(s == 15)
            def _():
                pltpu.sync_copy(acc.at[pl.ds(15 * STRIPE, LAST_ROWS)],
                                outlo_hbm.at[pl.ds(15 * STRIPE, LAST_ROWS)])

        @pl.when(c == 1)
        def _():
            @pl.when(s < 15)
            def _():
                pltpu.sync_copy(acc.at[pl.ds(r0, STRIPE)],
                                outhi_hbm.at[pl.ds(r0, STRIPE)])
            @pl.when(s == 15)
            def _():
                pltpu.sync_copy(acc.at[pl.ds(15 * STRIPE, LAST_ROWS)],
                                outhi_hbm.at[pl.ds(15 * STRIPE, LAST_ROWS)])

    return k(hs_lo, hs_hi, src2d, dst2d, zstripe)


def _front(x, cnt0, cnt1, winT, b_in, w1aT, w1bT):
    def body(x_ref, c0_ref, c1_ref, w_ref, b_ref, wa_ref, wb_ref,
             lo_ref, hi_ref, d_ref):
        deg = 1.0 + (jnp.sum(c0_ref[...], axis=1, keepdims=True)
                     + jnp.sum(c1_ref[...], axis=1, keepdims=True)) * (1.0 / HH)
        d = lax.rsqrt(deg)
        h0 = jnp.maximum(
            jnp.dot(x_ref[...], w_ref[...],
                    preferred_element_type=jnp.float32) + b_ref[...], 0.0)
        lo_ref[...] = jnp.dot(h0, wa_ref[...],
                              preferred_element_type=jnp.float32) * d
        hi_ref[...] = jnp.dot(h0, wb_ref[...],
                              preferred_element_type=jnp.float32) * d
        d_ref[...] = d

    full = lambda shape: pl.BlockSpec(shape, lambda i: (0, 0))
    return pl.pallas_call(
        body,
        grid=(NBLK,),
        in_specs=[
            pl.BlockSpec((BLK, D_IN), lambda i: (i, 0)),
            pl.BlockSpec((BLK, HH), lambda i: (i, 0)),
            pl.BlockSpec((BLK, HH), lambda i: (i, 0)),
            full((D_IN, H)), full((1, H)), full((H, HH)), full((H, HH)),
        ],
        out_specs=[
            pl.BlockSpec((BLK, HH), lambda i: (i, 0)),
            pl.BlockSpec((BLK, HH), lambda i: (i, 0)),
            pl.BlockSpec((BLK, 1), lambda i: (i, 0)),
        ],
        out_shape=[
            jax.ShapeDtypeStruct((N, HH), jnp.float32),
            jax.ShapeDtypeStruct((N, HH), jnp.float32),
            jax.ShapeDtypeStruct((N, 1), jnp.float32),
        ],
    )(x, cnt0, cnt1, winT, b_in, w1aT, w1bT)


def _bn_stats(s_lo, s_hi, hs_lo, hs_hi, dis, b_lo, b_hi):
    def body(sl_ref, sh_ref, hl_ref, hh_ref, d_ref, bl_ref, bh_ref,
             s1l_ref, s1h_ref, s2l_ref, s2h_ref):
        @pl.when(pl.program_id(0) == 0)
        def _():
            s1l_ref[...] = jnp.zeros_like(s1l_ref)
            s1h_ref[...] = jnp.zeros_like(s1h_ref)
            s2l_ref[...] = jnp.zeros_like(s2l_ref)
            s2h_ref[...] = jnp.zeros_like(s2h_ref)

        d = d_ref[...]
        tl = (sl_ref[...] + hl_ref[...]) * d + bl_ref[...]
        th = (sh_ref[...] + hh_ref[...]) * d + bh_ref[...]
        s1l_ref[...] += jnp.sum(tl, axis=0, keepdims=True)
        s1h_ref[...] += jnp.sum(th, axis=0, keepdims=True)
        s2l_ref[...] += jnp.sum(tl * tl, axis=0, keepdims=True)
        s2h_ref[...] += jnp.sum(th * th, axis=0, keepdims=True)

    blk = lambda: pl.BlockSpec((BLK, HH), lambda i: (i, 0))
    acc = lambda: pl.BlockSpec((1, HH), lambda i: (0, 0))
    return pl.pallas_call(
        body,
        grid=(NBLK,),
        in_specs=[blk(), blk(), blk(), blk(),
                  pl.BlockSpec((BLK, 1), lambda i: (i, 0)),
                  acc(), acc()],
        out_specs=[acc(), acc(), acc(), acc()],
        out_shape=[jax.ShapeDtypeStruct((1, HH), jnp.float32)] * 4,
    )(s_lo, s_hi, hs_lo, hs_hi, dis, b_lo, b_hi)


def _bn_apply(s_lo, s_hi, hs_lo, hs_hi, dis, b_lo, b_hi, g_lo, g_hi,
              be_lo, be_hi, stats, wq, last):
    s1l, s1h, s2l, s2h = stats

    def body(sl_ref, sh_ref, hl_ref, hh_ref, d_ref, bl_ref, bh_ref,
             gl_ref, gh_ref, bel_ref, beh_ref,
             s1l_ref, s1h_ref, s2l_ref, s2h_ref, *rest):
        if last:
            ol_ref, oh_ref = rest
        else:
            wll_ref, wlh_ref, whl_ref, whh_ref, ol_ref, oh_ref = rest
        d = d_ref[...]
        tl = (sl_ref[...] + hl_ref[...]) * d + bl_ref[...]
        th = (sh_ref[...] + hh_ref[...]) * d + bh_ref[...]
        inv_n = 1.0 / N
        ml = s1l_ref[...] * inv_n
        mh = s1h_ref[...] * inv_n
        vl = s2l_ref[...] * inv_n - ml * ml
        vh = s2h_ref[...] * inv_n - mh * mh
        scl = gl_ref[...] * lax.rsqrt(vl + EPS)
        sch = gh_ref[...] * lax.rsqrt(vh + EPS)
        rl = jnp.maximum(tl * scl + (bel_ref[...] - ml * scl), 0.0)
        rh = jnp.maximum(th * sch + (beh_ref[...] - mh * sch), 0.0)
        if last:
            ol_ref[...] = rl
            oh_ref[...] = rh
        else:
            ol_ref[...] = (jnp.dot(rl, wll_ref[...],
                                   preferred_element_type=jnp.float32)
                           + jnp.dot(rh, wlh_ref[...],
                                     preferred_element_type=jnp.float32)) * d
            oh_ref[...] = (jnp.dot(rl, whl_ref[...],
                                   preferred_element_type=jnp.float32)
                           + jnp.dot(rh, whh_ref[...],
                                     preferred_element_type=jnp.float32)) * d

    blk = lambda: pl.BlockSpec((BLK, HH), lambda i: (i, 0))
    row = lambda: pl.BlockSpec((1, HH), lambda i: (0, 0))
    wspec = lambda: pl.BlockSpec((HH, HH), lambda i: (0, 0))
    in_specs = [blk(), blk(), blk(), blk(),
                pl.BlockSpec((BLK, 1), lambda i: (i, 0)),
                row(), row(), row(), row(), row(), row(),
                row(), row(), row(), row()]
    args = [s_lo, s_hi, hs_lo, hs_hi, dis, b_lo, b_hi, g_lo, g_hi,
            be_lo, be_hi, s1l, s1h, s2l, s2h]
    if not last:
        in_specs += [wspec()] * 4
        args += list(wq)
    return pl.pallas_call(
        body,
        grid=(NBLK,),
        in_specs=in_specs,
        out_specs=[blk(), blk()],
        out_shape=[jax.ShapeDtypeStruct((N, HH), jnp.float32)] * 2,
    )(*args)


def _pool(h_lo, h_hi, starts):
    CH = 64

    def body(st_ref, hl_ref, hh_ref, ml_ref, mh_ref, xl_ref, xh_ref):
        g = pl.program_id(0)
        s0 = st_ref[g]
        e0 = st_ref[g + 1]
        n = e0 - s0
        nch = lax.div(n + (CH - 1), CH)

        def step(i, carry):
            sml, smh, mxl, mxh = carry
            base = s0 + i * CH
            b = jnp.minimum(base, N - CH)
            rl = hl_ref[pl.ds(b, CH), :]
            rh = hh_ref[pl.ds(b, CH), :]
            ridx = b + lax.broadcasted_iota(jnp.int32, (CH, 1), 0)
            m = (ridx >= base) & (ridx < e0)
            sml = sml + jnp.sum(jnp.where(m, rl, 0.0), axis=0, keepdims=True)
            smh = smh + jnp.sum(jnp.where(m, rh, 0.0), axis=0, keepdims=True)
            mxl = jnp.maximum(mxl, jnp.max(jnp.where(m, rl, -jnp.inf),
                                           axis=0, keepdims=True))
            mxh = jnp.maximum(mxh, jnp.max(jnp.where(m, rh, -jnp.inf),
                                           axis=0, keepdims=True))
            return sml, smh, mxl, mxh

        z = jnp.zeros((1, HH), jnp.float32)
        ninf = jnp.full((1, HH), -jnp.inf, jnp.float32)
        sml, smh, mxl, mxh = lax.fori_loop(0, nch, step, (z, z, ninf, ninf))
        cnt = jnp.maximum(n.astype(jnp.float32), 1.0)
        ml_ref[pl.ds(g, 1), :] = sml / cnt
        mh_ref[pl.ds(g, 1), :] = smh / cnt
        xl_ref[pl.ds(g, 1), :] = jnp.where(n > 0, mxl, 0.0)
        xh_ref[pl.ds(g, 1), :] = jnp.where(n > 0, mxh, 0.0)

    grid_spec = pltpu.PrefetchScalarGridSpec(
        num_scalar_prefetch=1,
        grid=(G,),
        in_specs=[pl.BlockSpec((N, HH), lambda g, st: (0, 0)),
                  pl.BlockSpec((N, HH), lambda g, st: (0, 0))],
        out_specs=[pl.BlockSpec((G, HH), lambda g, st: (0, 0))] * 4,
    )
    return pl.pallas_call(
        body,
        grid_spec=grid_spec,
        out_shape=[jax.ShapeDtypeStruct((G, HH), jnp.float32)] * 4,
    )(starts, h_lo, h_hi)


def _mlp(pooled, wc1T, bc1, wc2T, bc2):
    p0, p1, p2, p3 = pooled
    w0, w1, w2, w3 = wc1T

    def body(p0_ref, p1_ref, p2_ref, p3_ref, w0_ref, w1_ref, w2_ref, w3_ref,
             b1_ref, wc2_ref, b2_ref, o_ref):
        hc = (jnp.dot(p0_ref[...], w0_ref[...], preferred_element_type=jnp.float32)
              + jnp.dot(p1_ref[...], w1_ref[...], preferred_element_type=jnp.float32)
              + jnp.dot(p2_ref[...], w2_ref[...], preferred_element_type=jnp.float32)
              + jnp.dot(p3_ref[...], w3_ref[...], preferred_element_type=jnp.float32))
        hc = jnp.maximum(hc + b1_ref[...], 0.0)
        o_ref[...] = jnp.dot(hc, wc2_ref[...],
                             preferred_element_type=jnp.float32) + b2_ref[...]

    return pl.pallas_call(
        body,
        out_shape=jax.ShapeDtypeStruct((G, C), jnp.float32),
    )(p0, p1, p2, p3, w0, w1, w2, w3, bc1, wc2T, bc2)


def kernel(x, edge_index, batch, W_in, b_in, W1, b1, g1, be1, W2, b2, g2,
           be2, W3, b3, g3, be3, Wc1, bc1, Wc2, bc2):
    f32 = jnp.float32
    src = edge_index[0]
    dst = edge_index[1]
    pad = EP - E
    src2d = jnp.concatenate(
        [src, jnp.zeros((pad,), src.dtype)]).reshape(NROWS, CHUNK)
    dst2d = jnp.concatenate(
        [dst, jnp.full((pad,), N, dst.dtype)]).reshape(NROWS, CHUNK)
    ones32 = jnp.ones((CHUNK, HH), f32)
    zstripe = jnp.zeros((STRIPE, HH), f32)
    starts = jnp.searchsorted(
        batch, jnp.arange(G + 1, dtype=batch.dtype)).astype(jnp.int32)

    halves = lambda v: (v[:HH].reshape(1, HH).astype(f32),
                        v[HH:].reshape(1, HH).astype(f32))
    quarters = lambda W: (W[:HH, :HH].T, W[:HH, HH:].T,
                          W[HH:, :HH].T, W[HH:, HH:].T)
    winT = W_in.T
    b_in2 = b_in.reshape(1, H)
    w1aT, w1bT = W1[:HH, :].T, W1[HH:, :].T
    b1l, b1h = halves(b1); g1l, g1h = halves(g1); be1l, be1h = halves(be1)
    b2l, b2h = halves(b2); g2l, g2h = halves(g2); be2l, be2h = halves(be2)
    b3l, b3h = halves(b3); g3l, g3h = halves(g3); be3l, be3h = halves(be3)
    wq2 = quarters(W2)
    wq3 = quarters(W3)
    wc1T = tuple(Wc1[:, i * HH:(i + 1) * HH].T for i in range(4))
    bc1r = bc1.reshape(1, H)
    wc2T = Wc2.T
    bc2r = bc2.reshape(1, C)

    cnt0, cnt1 = _sc_degree(dst2d, ones32, zstripe)
    hs_lo, hs_hi, dis = _front(x, cnt0, cnt1, winT, b_in2, w1aT, w1bT)

    s_lo, s_hi = _sc_aggregate(hs_lo, hs_hi, src2d, dst2d, zstripe)
    st = _bn_stats(s_lo, s_hi, hs_lo, hs_hi, dis, b1l, b1h)
    hs_lo, hs_hi = _bn_apply(s_lo, s_hi, hs_lo, hs_hi, dis, b1l, b1h,
                             g1l, g1h, be1l, be1h, st, wq2, last=False)
    s_lo, s_hi = _sc_aggregate(hs_lo, hs_hi, src2d, dst2d, zstripe)
    st = _bn_stats(s_lo, s_hi, hs_lo, hs_hi, dis, b2l, b2h)
    hs_lo, hs_hi = _bn_apply(s_lo, s_hi, hs_lo, hs_hi, dis, b2l, b2h,
                             g2l, g2h, be2l, be2h, st, wq3, last=False)
    s_lo, s_hi = _sc_aggregate(hs_lo, hs_hi, src2d, dst2d, zstripe)
    st = _bn_stats(s_lo, s_hi, hs_lo, hs_hi, dis, b3l, b3h)
    h3_lo, h3_hi = _bn_apply(s_lo, s_hi, hs_lo, hs_hi, dis, b3l, b3h,
                             g3l, g3h, be3l, be3h, st, None, last=True)

    m_lo, m_hi, x_lo, x_hi = _pool(h3_lo, h3_hi, starts)
    return _mlp((m_lo, m_hi, x_lo, x_hi), wc1T, bc1r, wc2T, bc2r)


# speedup vs baseline: 15.7758x; 1.6172x over previous
import functools

import jax
import jax.numpy as jnp
from jax import lax
from jax.experimental import pallas as pl
from jax.experimental.pallas import tpu as pltpu
from jax.experimental.pallas import tpu_sc as plsc

N = 50000
E = 800000
D_IN = 128
H = 64
HH = H // 2
G = 256
C = 29

CHUNK = 128
NROWS = 6400
EP = NROWS * CHUNK
T_DEG = NROWS // 32
T_AGG = NROWS // 16
HHD = 16
N_ACC = 50048
STRIPE = N_ACC // 16
LAST_ROWS = N - 15 * STRIPE

SB = 8
NBUF = 4

BLK = 2000
NBLK = N // BLK
EPS = 1e-5


def _sc_mesh():
    return plsc.VectorSubcoreMesh(core_axis_name="c", subcore_axis_name="s")


_SC_PARAMS = pltpu.CompilerParams(use_tc_tiling_on_sc=False)


def _copy_out(cond_core, s, acc, out_hbm):
    @pl.when(cond_core)
    def _():
        @pl.when(s < 15)
        def _():
            pltpu.sync_copy(acc.at[pl.ds(s * STRIPE, STRIPE)],
                            out_hbm.at[pl.ds(s * STRIPE, STRIPE)])
        @pl.when(s == 15)
        def _():
            pltpu.sync_copy(acc.at[pl.ds(15 * STRIPE, LAST_ROWS)],
                            out_hbm.at[pl.ds(15 * STRIPE, LAST_ROWS)])


def _sc_degree(dst2d, ones32, zstripe):
    out_t = [jax.ShapeDtypeStruct((N, HHD), jnp.float32)] * 2

    @functools.partial(
        pl.kernel, mesh=_sc_mesh(), out_type=out_t,
        compiler_params=_SC_PARAMS,
        scratch_types=[
            pltpu.VMEM_SHARED((N_ACC, HHD), jnp.float32),
            pltpu.VMEM((CHUNK, HHD), jnp.float32),
            pltpu.VMEM((3, SB, CHUNK), jnp.int32),
        ] + [pltpu.SemaphoreType.DMA] * (NBUF + 1),
    )
    def k(dst_hbm, ones_hbm, z_hbm, cnt0_hbm, cnt1_hbm, acc, onesv, dstb,
          *sems):
        ssems = sems[:NBUF]
        isem = sems[NBUF]
        c = lax.axis_index("c")
        s = lax.axis_index("s")
        tb = (c * 16 + s) * T_DEG

        def wait_scatter(b):
            pltpu.make_async_copy(onesv, acc.at[dstb.at[0, 0]],
                                  ssems[b]).wait()

        pltpu.sync_copy(ones_hbm, onesv)
        pltpu.sync_copy(dst_hbm.at[pl.ds(tb, SB)], dstb.at[0])
        pltpu.async_copy(dst_hbm.at[pl.ds(tb + SB, SB)], dstb.at[1], isem)
        pltpu.sync_copy(z_hbm, acc.at[pl.ds(s * STRIPE, STRIPE)])
        plsc.subcore_barrier()

        @pl.loop(0, T_DEG, step=SB)
        def _(t0):
            q = lax.div(t0, SB)
            b0 = lax.rem(q, 3)
            b2 = lax.rem(q + 2, 3)
            for kk in range(SB):
                t = t0 + kk
                if kk == 0:
                    @pl.when(t0 > 0)
                    def _():
                        pltpu.make_async_copy(dst_hbm.at[pl.ds(tb, SB)],
                                              dstb.at[b0], isem).wait()
                @pl.when(t >= NBUF)
                def _():
                    wait_scatter(kk % NBUF)
                pltpu.async_copy(onesv, acc.at[dstb.at[b0, kk]],
                                 ssems[kk % NBUF], add=True)
                if kk == SB - 1:
                    @pl.when(t0 + 2 * SB < T_DEG)
                    def _():
                        pltpu.async_copy(
                            dst_hbm.at[pl.ds(tb + t0 + 2 * SB, SB)],
                            dstb.at[b2], isem)

        for b in range(NBUF):
            wait_scatter(b)
        plsc.subcore_barrier()
        _copy_out(c == 0, s, acc, cnt0_hbm)
        _copy_out(c == 1, s, acc, cnt1_hbm)

    return k(dst2d, ones32, zstripe)


def _sc_aggregate(hs_lo, hs_hi, src2d, dst2d, zstripe):
    out_t = [jax.ShapeDtypeStruct((N, HH), jnp.float32)] * 2

    @functools.partial(
        pl.kernel, mesh=_sc_mesh(), out_type=out_t,
        compiler_params=_SC_PARAMS,
        scratch_types=[
            pltpu.VMEM_SHARED((N_ACC, HH), jnp.float32),
            pltpu.VMEM((NBUF, CHUNK, HH), jnp.float32),
            pltpu.VMEM((3, SB, CHUNK), jnp.int32),
            pltpu.VMEM((3, SB, CHUNK), jnp.int32),
        ] + [pltpu.SemaphoreType.DMA] * (2 * NBUF + 2),
    )
    def k(hslo_hbm, hshi_hbm, src_hbm, dst_hbm, z_hbm,
          outlo_hbm, outhi_hbm, acc, msg, srcb, dstb, *sems):
        gsems = sems[:NBUF]
        ssems = sems[NBUF:2 * NBUF]
        isem_s = sems[2 * NBUF]
        isem_d = sems[2 * NBUF + 1]
        c = lax.axis_index("c")
        s = lax.axis_index("s")
        tb = s * T_AGG

        def issue_gather(buf, row_ref):
            @pl.when(c == 0)
            def _():
                pltpu.async_copy(hslo_hbm.at[row_ref], msg.at[buf],
                                 gsems[buf])
            @pl.when(c == 1)
            def _():
                pltpu.async_copy(hshi_hbm.at[row_ref], msg.at[buf],
                                 gsems[buf])

        def wait_gather(buf):
            pltpu.make_async_copy(z_hbm.at[srcb.at[0, 0]], msg.at[buf],
                                  gsems[buf]).wait()

        def wait_scatter(buf):
            pltpu.make_async_copy(msg.at[buf], acc.at[dstb.at[0, 0]],
                                  ssems[buf]).wait()

        pltpu.sync_copy(src_hbm.at[pl.ds(tb, SB)], srcb.at[0])
        pltpu.sync_copy(dst_hbm.at[pl.ds(tb, SB)], dstb.at[0])
        pltpu.async_copy(src_hbm.at[pl.ds(tb + SB, SB)], srcb.at[1], isem_s)
        pltpu.async_copy(dst_hbm.at[pl.ds(tb + SB, SB)], dstb.at[1], isem_d)
        pltpu.sync_copy(z_hbm, acc.at[pl.ds(s * STRIPE, STRIPE)])
        plsc.subcore_barrier()
        for kk in range(3):
            issue_gather(kk, srcb.at[0, kk])

        @pl.loop(0, T_AGG, step=SB)
        def _(t0):
            q = lax.div(t0, SB)
            b0 = lax.rem(q, 3)
            b1 = lax.rem(q + 1, 3)
            b2 = lax.rem(q + 2, 3)
            for kk in range(SB):
                t = t0 + kk
                buf = kk % NBUF
                m = t + 3
                mbuf = (kk + 3) % NBUF
                if kk == 5:
                    @pl.when(t0 + SB < T_AGG)
                    def _():
                        pltpu.make_async_copy(src_hbm.at[pl.ds(tb, SB)],
                                              srcb.at[b1], isem_s).wait()
                        pltpu.make_async_copy(dst_hbm.at[pl.ds(tb, SB)],
                                              dstb.at[b1], isem_d).wait()
                wait_gather(buf)
                @pl.when(m < T_AGG)
                def _():
                    @pl.when(m >= NBUF)
                    def _():
                        wait_scatter(mbuf)
                    if kk < 5:
                        issue_gather(mbuf, srcb.at[b0, kk + 3])
                    else:
                        issue_gather(mbuf, srcb.at[b1, kk - 5])
                pltpu.async_copy(msg.at[buf], acc.at[dstb.at[b0, kk]],
                                 ssems[buf], add=True)
                if kk == 4:
                    @pl.when(t0 + 2 * SB < T_AGG)
                    def _():
                        pltpu.async_copy(
                            src_hbm.at[pl.ds(tb + t0 + 2 * SB, SB)],
                            srcb.at[b2], isem_s)
                        pltpu.async_copy(
                            dst_hbm.at[pl.ds(tb + t0 + 2 * SB, SB)],
                            dstb.at[b2], isem_d)

        for b in range(NBUF):
            wait_scatter(b)
        plsc.subcore_barrier()
        _copy_out(c == 0, s, acc, outlo_hbm)
        _copy_out(c == 1, s, acc, outhi_hbm)

    return k(hs_lo, hs_hi, src2d, dst2d, zstripe)


def _front(x, cnt0, cnt1, winT, b_in, w1aT, w1bT):
    def body(x_ref, c0_ref, c1_ref, w_ref, b_ref, wa_ref, wb_ref,
             lo_ref, hi_ref, d_ref):
        deg = 1.0 + (jnp.sum(c0_ref[...], axis=1, keepdims=True)
                     + jnp.sum(c1_ref[...], axis=1, keepdims=True)) * (1.0 / HHD)
        d = lax.rsqrt(deg)
        h0 = jnp.maximum(
            jnp.dot(x_ref[...], w_ref[...],
                    preferred_element_type=jnp.float32) + b_ref[...], 0.0)
        lo_ref[...] = jnp.dot(h0, wa_ref[...],
                              preferred_element_type=jnp.float32) * d
        hi_ref[...] = jnp.dot(h0, wb_ref[...],
                              preferred_element_type=jnp.float32) * d
        d_ref[...] = d

    full = lambda shape: pl.BlockSpec(shape, lambda i: (0, 0))
    return pl.pallas_call(
        body,
        grid=(NBLK,),
        in_specs=[
            pl.BlockSpec((BLK, D_IN), lambda i: (i, 0)),
            pl.BlockSpec((BLK, HHD), lambda i: (i, 0)),
            pl.BlockSpec((BLK, HHD), lambda i: (i, 0)),
            full((D_IN, H)), full((1, H)), full((H, HH)), full((H, HH)),
        ],
        out_specs=[
            pl.BlockSpec((BLK, HH), lambda i: (i, 0)),
            pl.BlockSpec((BLK, HH), lambda i: (i, 0)),
            pl.BlockSpec((BLK, 1), lambda i: (i, 0)),
        ],
        out_shape=[
            jax.ShapeDtypeStruct((N, HH), jnp.float32),
            jax.ShapeDtypeStruct((N, HH), jnp.float32),
            jax.ShapeDtypeStruct((N, 1), jnp.float32),
        ],
    )(x, cnt0, cnt1, winT, b_in, w1aT, w1bT)


def _bn_stats(s_lo, s_hi, hs_lo, hs_hi, dis, b_lo, b_hi):
    def body(sl_ref, sh_ref, hl_ref, hh_ref, d_ref, bl_ref, bh_ref,
             s1l_ref, s1h_ref, s2l_ref, s2h_ref):
        @pl.when(pl.program_id(0) == 0)
        def _():
            s1l_ref[...] = jnp.zeros_like(s1l_ref)
            s1h_ref[...] = jnp.zeros_like(s1h_ref)
            s2l_ref[...] = jnp.zeros_like(s2l_ref)
            s2h_ref[...] = jnp.zeros_like(s2h_ref)

        d = d_ref[...]
        tl = (sl_ref[...] + hl_ref[...]) * d + bl_ref[...]
        th = (sh_ref[...] + hh_ref[...]) * d + bh_ref[...]
        s1l_ref[...] += jnp.sum(tl, axis=0, keepdims=True)
        s1h_ref[...] += jnp.sum(th, axis=0, keepdims=True)
        s2l_ref[...] += jnp.sum(tl * tl, axis=0, keepdims=True)
        s2h_ref[...] += jnp.sum(th * th, axis=0, keepdims=True)

    blk = lambda: pl.BlockSpec((BLK, HH), lambda i: (i, 0))
    acc = lambda: pl.BlockSpec((1, HH), lambda i: (0, 0))
    return pl.pallas_call(
        body,
        grid=(NBLK,),
        in_specs=[blk(), blk(), blk(), blk(),
                  pl.BlockSpec((BLK, 1), lambda i: (i, 0)),
                  acc(), acc()],
        out_specs=[acc(), acc(), acc(), acc()],
        out_shape=[jax.ShapeDtypeStruct((1, HH), jnp.float32)] * 4,
    )(s_lo, s_hi, hs_lo, hs_hi, dis, b_lo, b_hi)


def _bn_apply(s_lo, s_hi, hs_lo, hs_hi, dis, b_lo, b_hi, g_lo, g_hi,
              be_lo, be_hi, stats, wq, last):
    s1l, s1h, s2l, s2h = stats

    def body(sl_ref, sh_ref, hl_ref, hh_ref, d_ref, bl_ref, bh_ref,
             gl_ref, gh_ref, bel_ref, beh_ref,
             s1l_ref, s1h_ref, s2l_ref, s2h_ref, *rest):
        if last:
            ol_ref, oh_ref = rest
        else:
            wll_ref, wlh_ref, whl_ref, whh_ref, ol_ref, oh_ref = rest
        d = d_ref[...]
        tl = (sl_ref[...] + hl_ref[...]) * d + bl_ref[...]
        th = (sh_ref[...] + hh_ref[...]) * d + bh_ref[...]
        inv_n = 1.0 / N
        ml = s1l_ref[...] * inv_n
        mh = s1h_ref[...] * inv_n
        vl = s2l_ref[...] * inv_n - ml * ml
        vh = s2h_ref[...] * inv_n - mh * mh
        scl = gl_ref[...] * lax.rsqrt(vl + EPS)
        sch = gh_ref[...] * lax.rsqrt(vh + EPS)
        rl = jnp.maximum(tl * scl + (bel_ref[...] - ml * scl), 0.0)
        rh = jnp.maximum(th * sch + (beh_ref[...] - mh * sch), 0.0)
        if last:
            ol_ref[...] = rl
            oh_ref[...] = rh
        else:
            ol_ref[...] = (jnp.dot(rl, wll_ref[...],
                                   preferred_element_type=jnp.float32)
                           + jnp.dot(rh, wlh_ref[...],
                                     preferred_element_type=jnp.float32)) * d
            oh_ref[...] = (jnp.dot(rl, whl_ref[...],
                                   preferred_element_type=jnp.float32)
                           + jnp.dot(rh, whh_ref[...],
                                     preferred_element_type=jnp.float32)) * d

    blk = lambda: pl.BlockSpec((BLK, HH), lambda i: (i, 0))
    row = lambda: pl.BlockSpec((1, HH), lambda i: (0, 0))
    wspec = lambda: pl.BlockSpec((HH, HH), lambda i: (0, 0))
    in_specs = [blk(), blk(), blk(), blk(),
                pl.BlockSpec((BLK, 1), lambda i: (i, 0)),
                row(), row(), row(), row(), row(), row(),
                row(), row(), row(), row()]
    args = [s_lo, s_hi, hs_lo, hs_hi, dis, b_lo, b_hi, g_lo, g_hi,
            be_lo, be_hi, s1l, s1h, s2l, s2h]
    if not last:
        in_specs += [wspec()] * 4
        args += list(wq)
    return pl.pallas_call(
        body,
        grid=(NBLK,),
        in_specs=in_specs,
        out_specs=[blk(), blk()],
        out_shape=[jax.ShapeDtypeStruct((N, HH), jnp.float32)] * 2,
    )(*args)


def _pool(h_lo, h_hi, starts):
    CH = 64

    def body(st_ref, hl_ref, hh_ref, ml_ref, mh_ref, xl_ref, xh_ref):
        g = pl.program_id(0)
        s0 = st_ref[g]
        e0 = st_ref[g + 1]
        n = e0 - s0
        nch = lax.div(n + (CH - 1), CH)

        def step(i, carry):
            sml, smh, mxl, mxh = carry
            base = s0 + i * CH
            b = jnp.minimum(base, N - CH)
            rl = hl_ref[pl.ds(b, CH), :]
            rh = hh_ref[pl.ds(b, CH), :]
            ridx = b + lax.broadcasted_iota(jnp.int32, (CH, 1), 0)
            m = (ridx >= base) & (ridx < e0)
            sml = sml + jnp.sum(jnp.where(m, rl, 0.0), axis=0, keepdims=True)
            smh = smh + jnp.sum(jnp.where(m, rh, 0.0), axis=0, keepdims=True)
            mxl = jnp.maximum(mxl, jnp.max(jnp.where(m, rl, -jnp.inf),
                                           axis=0, keepdims=True))
            mxh = jnp.maximum(mxh, jnp.max(jnp.where(m, rh, -jnp.inf),
                                           axis=0, keepdims=True))
            return sml, smh, mxl, mxh

        z = jnp.zeros((1, HH), jnp.float32)
        ninf = jnp.full((1, HH), -jnp.inf, jnp.float32)
        sml, smh, mxl, mxh = lax.fori_loop(0, nch, step, (z, z, ninf, ninf))
        cnt = jnp.maximum(n.astype(jnp.float32), 1.0)
        ml_ref[pl.ds(g, 1), :] = sml / cnt
        mh_ref[pl.ds(g, 1), :] = smh / cnt
        xl_ref[pl.ds(g, 1), :] = jnp.where(n > 0, mxl, 0.0)
        xh_ref[pl.ds(g, 1), :] = jnp.where(n > 0, mxh, 0.0)

    grid_spec = pltpu.PrefetchScalarGridSpec(
        num_scalar_prefetch=1,
        grid=(G,),
        in_specs=[pl.BlockSpec((N, HH), lambda g, st: (0, 0)),
                  pl.BlockSpec((N, HH), lambda g, st: (0, 0))],
        out_specs=[pl.BlockSpec((G, HH), lambda g, st: (0, 0))] * 4,
    )
    return pl.pallas_call(
        body,
        grid_spec=grid_spec,
        out_shape=[jax.ShapeDtypeStruct((G, HH), jnp.float32)] * 4,
    )(starts, h_lo, h_hi)


def _mlp(pooled, wc1T, bc1, wc2T, bc2):
    p0, p1, p2, p3 = pooled
    w0, w1, w2, w3 = wc1T

    def body(p0_ref, p1_ref, p2_ref, p3_ref, w0_ref, w1_ref, w2_ref, w3_ref,
             b1_ref, wc2_ref, b2_ref, o_ref):
        hc = (jnp.dot(p0_ref[...], w0_ref[...], preferred_element_type=jnp.float32)
              + jnp.dot(p1_ref[...], w1_ref[...], preferred_element_type=jnp.float32)
              + jnp.dot(p2_ref[...], w2_ref[...], preferred_element_type=jnp.float32)
              + jnp.dot(p3_ref[...], w3_ref[...], preferred_element_type=jnp.float32))
        hc = jnp.maximum(hc + b1_ref[...], 0.0)
        o_ref[...] = jnp.dot(hc, wc2_ref[...],
                             preferred_element_type=jnp.float32) + b2_ref[...]

    return pl.pallas_call(
        body,
        out_shape=jax.ShapeDtypeStruct((G, C), jnp.float32),
    )(p0, p1, p2, p3, w0, w1, w2, w3, bc1, wc2T, bc2)


def kernel(x, edge_index, batch, W_in, b_in, W1, b1, g1, be1, W2, b2, g2,
           be2, W3, b3, g3, be3, Wc1, bc1, Wc2, bc2):
    f32 = jnp.float32
    src = edge_index[0]
    dst = edge_index[1]
    pad = EP - E
    src2d = jnp.concatenate(
        [src, jnp.zeros((pad,), src.dtype)]).reshape(NROWS, CHUNK)
    dst2d = jnp.concatenate(
        [dst, jnp.full((pad,), N, dst.dtype)]).reshape(NROWS, CHUNK)
    ones16 = jnp.ones((CHUNK, HHD), f32)
    zstripe16 = jnp.zeros((STRIPE, HHD), f32)
    zstripe = jnp.zeros((STRIPE, HH), f32)
    starts = jnp.searchsorted(
        batch, jnp.arange(G + 1, dtype=batch.dtype)).astype(jnp.int32)

    halves = lambda v: (v[:HH].reshape(1, HH).astype(f32),
                        v[HH:].reshape(1, HH).astype(f32))
    quarters = lambda W: (W[:HH, :HH].T, W[:HH, HH:].T,
                          W[HH:, :HH].T, W[HH:, HH:].T)
    winT = W_in.T
    b_in2 = b_in.reshape(1, H)
    w1aT, w1bT = W1[:HH, :].T, W1[HH:, :].T
    b1l, b1h = halves(b1); g1l, g1h = halves(g1); be1l, be1h = halves(be1)
    b2l, b2h = halves(b2); g2l, g2h = halves(g2); be2l, be2h = halves(be2)
    b3l, b3h = halves(b3); g3l, g3h = halves(g3); be3l, be3h = halves(be3)
    wq2 = quarters(W2)
    wq3 = quarters(W3)
    wc1T = tuple(Wc1[:, i * HH:(i + 1) * HH].T for i in range(4))
    bc1r = bc1.reshape(1, H)
    wc2T = Wc2.T
    bc2r = bc2.reshape(1, C)

    cnt0, cnt1 = _sc_degree(dst2d, ones16, zstripe16)
    hs_lo, hs_hi, dis = _front(x, cnt0, cnt1, winT, b_in2, w1aT, w1bT)

    s_lo, s_hi = _sc_aggregate(hs_lo, hs_hi, src2d, dst2d, zstripe)
    st = _bn_stats(s_lo, s_hi, hs_lo, hs_hi, dis, b1l, b1h)
    hs_lo, hs_hi = _bn_apply(s_lo, s_hi, hs_lo, hs_hi, dis, b1l, b1h,
                             g1l, g1h, be1l, be1h, st, wq2, last=False)
    s_lo, s_hi = _sc_aggregate(hs_lo, hs_hi, src2d, dst2d, zstripe)
    st = _bn_stats(s_lo, s_hi, hs_lo, hs_hi, dis, b2l, b2h)
    hs_lo, hs_hi = _bn_apply(s_lo, s_hi, hs_lo, hs_hi, dis, b2l, b2h,
                             g2l, g2h, be2l, be2h, st, wq3, last=False)
    s_lo, s_hi = _sc_aggregate(hs_lo, hs_hi, src2d, dst2d, zstripe)
    st = _bn_stats(s_lo, s_hi, hs_lo, hs_hi, dis, b3l, b3h)
    h3_lo, h3_hi = _bn_apply(s_lo, s_hi, hs_lo, hs_hi, dis, b3l, b3h,
                             g3l, g3h, be3l, be3h, st, None, last=True)

    m_lo, m_hi, x_lo, x_hi = _pool(h3_lo, h3_hi, starts)
    return _mlp((m_lo, m_hi, x_lo, x_hi), wc1T, bc1r, wc2T, bc2r)
